# baseline (speedup 1.0000x reference)
import functools

import jax
import jax.numpy as jnp
from jax import lax
from jax.experimental import pallas as pl
from jax.experimental.pallas import tpu as pltpu
from jax.experimental.pallas import tpu_sc as plsc

_E = 320000
_D = 128
_CH = 128
_NCH = _E // _CH
_NC = 2
_NS = 16
_NW = _NC * _NS
_TROWS = 224


_SC = 256
_NSC = _E // _SC
_NQ = _SC // _CH
_MAXI = (_NSC + _NW - 1) // _NW + 1


_W0N, _W1N, _W2N = 7 * _D, 8 * _D, 4 * _D


def _sc_gather(a3, w0f, w1f, w2f):
    mesh = plsc.VectorSubcoreMesh(core_axis_name="c", subcore_axis_name="s")

    @functools.partial(
        pl.kernel,
        out_type=jax.ShapeDtypeStruct((_E, _D), jnp.float32),
        mesh=mesh,
        scratch_types=[
            pltpu.VMEM((3 * _SC,), jnp.int32),
            pltpu.VMEM((3 * _SC,), jnp.int32),
            pltpu.VMEM((_CH,), jnp.int32),
            pltpu.VMEM((_CH,), jnp.int32),
            pltpu.VMEM((_CH,), jnp.int32),
            pltpu.VMEM((_CH,), jnp.int32),
            pltpu.VMEM((_SC, _D), jnp.float32),
            pltpu.VMEM((_SC, _D), jnp.float32),
            pltpu.VMEM((_W0N + _W1N + _W2N,), jnp.float32),
            pltpu.VMEM((_D,), jnp.float32),
            pltpu.VMEM_SHARED((_TROWS, _D), jnp.float32),
            pltpu.SemaphoreType.DMA,
            pltpu.SemaphoreType.DMA,
            pltpu.SemaphoreType.DMA,
            pltpu.SemaphoreType.DMA,
            pltpu.SemaphoreType.DMA,
        ],
    )
    def k(a3_hbm, w0_hbm, w1_hbm, w2_hbm, out_hbm,
          a3v0, a3v1, cx00, cx01, cx10, cx11, rows0, rows1, wv, rowbuf, t_sh,
          semi0, semi1, semo0, semo1, semg):
        a3v = (a3v0, a3v1)
        cx = ((cx00, cx01), (cx10, cx11))
        rows = (rows0, rows1)
        semi = (semi0, semi1)
        semo = (semo0, semo1)

        cid = lax.axis_index("c")
        sid = lax.axis_index("s")
        wid = sid * _NC + cid

        pltpu.sync_copy(w0_hbm, wv.at[pl.ds(0, _W0N)])
        pltpu.sync_copy(w1_hbm, wv.at[pl.ds(_W0N, _W1N)])
        pltpu.sync_copy(w2_hbm, wv.at[pl.ds(_W0N + _W1N, _W2N)])
        for kk in range(_TROWS // _NS):
            r = sid * (_TROWS // _NS) + kk
            o0 = (r // 32) * _D
            o1 = _W0N + ((r // 4) % 8) * _D
            o2 = _W0N + _W1N + (r % 4) * _D
            for g in range(_D // 16):
                rowbuf[pl.ds(16 * g, 16)] = (
                    wv[pl.ds(o0 + 16 * g, 16)]
                    + wv[pl.ds(o1 + 16 * g, 16)]
                    + wv[pl.ds(o2 + 16 * g, 16)])
            pltpu.sync_copy(rowbuf, t_sh.at[r])

        plsc.subcore_barrier()

        niter = (_NSC - wid + _NW - 1) // _NW

        pltpu.async_copy(a3_hbm.at[pl.ds(wid * (3 * _SC), 3 * _SC)],
                         a3v[0], semi[0])

        def body(j, carry):
            for b in range(2):
                i = j * 2 + b

                @pl.when(i < niter)
                def _do():
                    s = wid + i * _NW
                    base = s * _SC
                    pltpu.make_async_copy(
                        a3_hbm.at[pl.ds(0, 3 * _SC)], a3v[b], semi[b]).wait()
                    @pl.when(i + 1 < niter)
                    def _pf():
                        nbase = (wid + (i + 1) * _NW) * (3 * _SC)
                        pltpu.async_copy(
                            a3_hbm.at[pl.ds(nbase, 3 * _SC)],
                            a3v[1 - b], semi[1 - b])
                    av = a3v[b]
                    for q in range(_NQ):
                        cq = cx[b][q]
                        for t in range(_CH // 16):
                            o = q * _CH + 16 * t
                            s0 = pl.ds(o, 16)
                            s1 = pl.ds(o + _SC, 16)
                            s2 = pl.ds(o + 2 * _SC, 16)
                            cq[pl.ds(16 * t, 16)] = (
                                av[s0] * 32 + av[s1] * 4 + av[s2])
                    @pl.when(i >= 2)
                    def _drain():
                        pltpu.make_async_copy(
                            rows[b], out_hbm.at[pl.ds(0, _SC)], semo[b]).wait()
                    for q in range(_NQ):
                        pltpu.async_copy(
                            t_sh.at[cx[b][q]],
                            rows[b].at[pl.ds(q * _CH, _CH)], semg)
                    for q in range(_NQ):
                        pltpu.make_async_copy(
                            t_sh.at[cx[b][q]],
                            rows[b].at[pl.ds(q * _CH, _CH)], semg).wait()
                    pltpu.async_copy(rows[b], out_hbm.at[pl.ds(base, _SC)],
                                     semo[b])
            return carry

        lax.fori_loop(0, _MAXI // 2, body, 0)

        pltpu.make_async_copy(rows[0], out_hbm.at[pl.ds(0, _SC)], semo[0]).wait()
        pltpu.make_async_copy(rows[1], out_hbm.at[pl.ds(0, _SC)], semo[1]).wait()

    return k(a3, w0f, w1f, w2f)


def kernel(edge_attr, W0, W1, W2):
    ea = edge_attr.astype(jnp.int32)
    a3 = ea.reshape(_NSC, _SC, 3).transpose(0, 2, 1).reshape(-1)
    return _sc_gather(a3, W0.reshape(-1), W1.reshape(-1), W2.reshape(-1))


# speedup vs baseline: 1.0215x; 1.0215x over previous
import functools

import jax
import jax.numpy as jnp
from jax import lax
from jax.experimental import pallas as pl
from jax.experimental.pallas import tpu as pltpu
from jax.experimental.pallas import tpu_sc as plsc

_E = 320000
_D = 128
_CH = 128
_NCH = _E // _CH
_NC = 2
_NS = 16
_NW = _NC * _NS
_TROWS = 224


def _table_body(w0_ref, w1_ref, w2_ref, t_ref):
    r = lax.broadcasted_iota(jnp.int32, (_TROWS, 8), 0)
    k = lax.broadcasted_iota(jnp.int32, (_TROWS, 8), 1)
    a0 = (r // 32 == k).astype(jnp.float32)[:, :7]
    a1 = ((r // 4) % 8 == k).astype(jnp.float32)
    a2 = (r % 4 == k).astype(jnp.float32)[:, :4]
    t_ref[...] = (
        jnp.dot(a0, w0_ref[...], preferred_element_type=jnp.float32)
        + jnp.dot(a1, w1_ref[...], preferred_element_type=jnp.float32)
        + jnp.dot(a2, w2_ref[...], preferred_element_type=jnp.float32)
    )


def _build_table(w0, w1, w2):
    return pl.pallas_call(
        _table_body,
        out_shape=jax.ShapeDtypeStruct((_TROWS, _D), jnp.float32),
    )(w0, w1, w2)


_SC = 256
_NSC = _E // _SC
_NQ = _SC // _CH
_MAXI = (_NSC + _NW - 1) // _NW + 1


def _sc_gather(a3, table):
    mesh = plsc.VectorSubcoreMesh(core_axis_name="c", subcore_axis_name="s")

    @functools.partial(
        pl.kernel,
        out_type=jax.ShapeDtypeStruct((_E, _D), jnp.float32),
        mesh=mesh,
        scratch_types=[
            pltpu.VMEM((3 * _SC,), jnp.int32),
            pltpu.VMEM((3 * _SC,), jnp.int32),
            pltpu.VMEM((_CH,), jnp.int32),
            pltpu.VMEM((_CH,), jnp.int32),
            pltpu.VMEM((_CH,), jnp.int32),
            pltpu.VMEM((_CH,), jnp.int32),
            pltpu.VMEM((_SC, _D), jnp.float32),
            pltpu.VMEM((_SC, _D), jnp.float32),
            pltpu.VMEM((_TROWS, _D), jnp.float32),
            pltpu.VMEM_SHARED((_TROWS, _D), jnp.float32),
            pltpu.SemaphoreType.DMA,
            pltpu.SemaphoreType.DMA,
            pltpu.SemaphoreType.DMA,
            pltpu.SemaphoreType.DMA,
            pltpu.SemaphoreType.DMA,
        ],
    )
    def k(a3_hbm, t_hbm, out_hbm,
          a3v0, a3v1, cx00, cx01, cx10, cx11, rows0, rows1, t_v, t_sh,
          semi0, semi1, semo0, semo1, semg):
        a3v = (a3v0, a3v1)
        cx = ((cx00, cx01), (cx10, cx11))
        rows = (rows0, rows1)
        semi = (semi0, semi1)
        semo = (semo0, semo1)

        cid = lax.axis_index("c")
        sid = lax.axis_index("s")
        wid = sid * _NC + cid

        niter = (_NSC - wid + _NW - 1) // _NW

        pltpu.async_copy(a3_hbm.at[pl.ds(wid * (3 * _SC), 3 * _SC)],
                         a3v[0], semi[0])

        @pl.when(sid == 0)
        def _stage():
            pltpu.sync_copy(t_hbm, t_v)
            pltpu.sync_copy(t_v, t_sh)

        plsc.subcore_barrier()

        def body(j, carry):
            for b in range(2):
                i = j * 2 + b

                @pl.when(i < niter)
                def _do():
                    s = wid + i * _NW
                    base = s * _SC
                    pltpu.make_async_copy(
                        a3_hbm.at[pl.ds(0, 3 * _SC)], a3v[b], semi[b]).wait()
                    @pl.when(i + 1 < niter)
                    def _pf():
                        nbase = (wid + (i + 1) * _NW) * (3 * _SC)
                        pltpu.async_copy(
                            a3_hbm.at[pl.ds(nbase, 3 * _SC)],
                            a3v[1 - b], semi[1 - b])
                    av = a3v[b]
                    for q in range(_NQ):
                        cq = cx[b][q]
                        for t in range(_CH // 16):
                            o = q * _CH + 16 * t
                            s0 = pl.ds(o, 16)
                            s1 = pl.ds(o + _SC, 16)
                            s2 = pl.ds(o + 2 * _SC, 16)
                            cq[pl.ds(16 * t, 16)] = (
                                av[s0] * 32 + av[s1] * 4 + av[s2])
                    @pl.when(i >= 2)
                    def _drain():
                        pltpu.make_async_copy(
                            rows[b], out_hbm.at[pl.ds(0, _SC)], semo[b]).wait()
                    for q in range(_NQ):
                        pltpu.async_copy(
                            t_sh.at[cx[b][q]],
                            rows[b].at[pl.ds(q * _CH, _CH)], semg)
                    for q in range(_NQ):
                        pltpu.make_async_copy(
                            t_sh.at[cx[b][q]],
                            rows[b].at[pl.ds(q * _CH, _CH)], semg).wait()
                    pltpu.async_copy(rows[b], out_hbm.at[pl.ds(base, _SC)],
                                     semo[b])
            return carry

        lax.fori_loop(0, _MAXI // 2, body, 0)

        pltpu.make_async_copy(rows[0], out_hbm.at[pl.ds(0, _SC)], semo[0]).wait()
        pltpu.make_async_copy(rows[1], out_hbm.at[pl.ds(0, _SC)], semo[1]).wait()

    return k(a3, table)


def kernel(edge_attr, W0, W1, W2):
    ea = edge_attr.astype(jnp.int32)
    a3 = ea.reshape(_NSC, _SC, 3).transpose(0, 2, 1).reshape(-1)
    table = _build_table(W0, W1, W2)
    return _sc_gather(a3, table)


# speedup vs baseline: 1.0422x; 1.0203x over previous
import functools

import jax
import jax.numpy as jnp
from jax import lax
from jax.experimental import pallas as pl
from jax.experimental.pallas import tpu as pltpu
from jax.experimental.pallas import tpu_sc as plsc

_E = 320000
_D = 128
_CH = 128
_NCH = _E // _CH
_NC = 2
_NS = 16
_NW = _NC * _NS
_TROWS = 224


def _table_body(w0_ref, w1_ref, w2_ref, t_ref):
    r = lax.broadcasted_iota(jnp.int32, (_TROWS, 8), 0)
    k = lax.broadcasted_iota(jnp.int32, (_TROWS, 8), 1)
    a0 = (r // 32 == k).astype(jnp.float32)[:, :7]
    a1 = ((r // 4) % 8 == k).astype(jnp.float32)
    a2 = (r % 4 == k).astype(jnp.float32)[:, :4]
    t_ref[...] = (
        jnp.dot(a0, w0_ref[...], preferred_element_type=jnp.float32)
        + jnp.dot(a1, w1_ref[...], preferred_element_type=jnp.float32)
        + jnp.dot(a2, w2_ref[...], preferred_element_type=jnp.float32)
    )


def _build_table(w0, w1, w2):
    return pl.pallas_call(
        _table_body,
        out_shape=jax.ShapeDtypeStruct((_TROWS, _D), jnp.float32),
    )(w0, w1, w2)


_SC = 256
_NSC = _E // _SC
_NQ = _SC // _CH
_MAXI = (_NSC + _NW - 1) // _NW + 1


def _sc_gather(a3, table):
    mesh = plsc.VectorSubcoreMesh(core_axis_name="c", subcore_axis_name="s")

    @functools.partial(
        pl.kernel,
        out_type=jax.ShapeDtypeStruct((_E, _D), jnp.float32),
        mesh=mesh,
        scratch_types=[
            pltpu.VMEM((3 * _SC,), jnp.int32),
            pltpu.VMEM((3 * _SC,), jnp.int32),
            pltpu.VMEM((_CH,), jnp.int32),
            pltpu.VMEM((_CH,), jnp.int32),
            pltpu.VMEM((_CH,), jnp.int32),
            pltpu.VMEM((_CH,), jnp.int32),
            pltpu.VMEM((_SC, _D), jnp.float32),
            pltpu.VMEM((_SC, _D), jnp.float32),
            pltpu.VMEM((_TROWS, _D), jnp.float32),
            pltpu.VMEM_SHARED((_TROWS, _D), jnp.float32),
            pltpu.SemaphoreType.DMA,
            pltpu.SemaphoreType.DMA,
            pltpu.SemaphoreType.DMA,
            pltpu.SemaphoreType.DMA,
            pltpu.SemaphoreType.DMA,
        ],
    )
    def k(a3_hbm, t_hbm, out_hbm,
          a3v0, a3v1, cx00, cx01, cx10, cx11, rows0, rows1, t_v, t_sh,
          semi0, semi1, semo0, semo1, semg):
        a3v = (a3v0, a3v1)
        cx = ((cx00, cx01), (cx10, cx11))
        rows = (rows0, rows1)
        semi = (semi0, semi1)
        semo = (semo0, semo1)

        cid = lax.axis_index("c")
        sid = lax.axis_index("s")
        wid = sid * _NC + cid

        niter = (_NSC - wid + _NW - 1) // _NW

        pltpu.async_copy(a3_hbm.at[pl.ds(wid * (3 * _SC), 3 * _SC)],
                         a3v[0], semi[0])

        @pl.when(sid == 0)
        def _stage():
            pltpu.sync_copy(t_hbm, t_sh)

        plsc.subcore_barrier()

        def body(j, carry):
            for b in range(2):
                i = j * 2 + b

                @pl.when(i < niter)
                def _do():
                    s = wid + i * _NW
                    base = s * _SC
                    pltpu.make_async_copy(
                        a3_hbm.at[pl.ds(0, 3 * _SC)], a3v[b], semi[b]).wait()
                    @pl.when(i + 1 < niter)
                    def _pf():
                        nbase = (wid + (i + 1) * _NW) * (3 * _SC)
                        pltpu.async_copy(
                            a3_hbm.at[pl.ds(nbase, 3 * _SC)],
                            a3v[1 - b], semi[1 - b])
                    av = a3v[b]
                    for q in range(_NQ):
                        cq = cx[b][q]
                        for t in range(_CH // 16):
                            o = q * _CH + 16 * t
                            s0 = pl.ds(o, 16)
                            s1 = pl.ds(o + _SC, 16)
                            s2 = pl.ds(o + 2 * _SC, 16)
                            cq[pl.ds(16 * t, 16)] = (
                                av[s0] * 32 + av[s1] * 4 + av[s2])
                    @pl.when(i >= 2)
                    def _drain():
                        pltpu.make_async_copy(
                            rows[b], out_hbm.at[pl.ds(0, _SC)], semo[b]).wait()
                    for q in range(_NQ):
                        pltpu.async_copy(
                            t_sh.at[cx[b][q]],
                            rows[b].at[pl.ds(q * _CH, _CH)], semg)
                    for q in range(_NQ):
                        pltpu.make_async_copy(
                            t_sh.at[cx[b][q]],
                            rows[b].at[pl.ds(q * _CH, _CH)], semg).wait()
                    pltpu.async_copy(rows[b], out_hbm.at[pl.ds(base, _SC)],
                                     semo[b])
            return carry

        lax.fori_loop(0, _MAXI // 2, body, 0)

        pltpu.make_async_copy(rows[0], out_hbm.at[pl.ds(0, _SC)], semo[0]).wait()
        pltpu.make_async_copy(rows[1], out_hbm.at[pl.ds(0, _SC)], semo[1]).wait()

    return k(a3, table)


def kernel(edge_attr, W0, W1, W2):
    ea = edge_attr.astype(jnp.int32)
    a3 = ea.reshape(_NSC, _SC, 3).transpose(0, 2, 1).reshape(-1)
    table = _build_table(W0, W1, W2)
    return _sc_gather(a3, table)
